# Initial kernel scaffold; baseline (speedup 1.0000x reference)
import jax
import jax.numpy as jnp
from jax.experimental import pallas as pl


def kernel(x, adj, W1, b1, theta0, theta1, Wt, bt, Wr, ln_gamma, ln_beta):
    raise NotImplementedError("write your pallas kernel here")


# speedup vs baseline: 6899.6433x; 6899.6433x over previous
import jax
import jax.numpy as jnp
from jax.experimental import pallas as pl
from jax.experimental.pallas import tpu as pltpu


def _encoder_body(xt_ref, adj_ref, adjT_ref, w1_ref, b1_ref, th0_ref, th1_ref,
                  wtT_ref, bt_ref, wr_ref, gamma_ref, beta_ref,
                  out_ref, sp_pad, featbuf):
    B, T, N, D = xt_ref.shape
    F = th0_ref.shape[1]
    dot = lambda a, b: jnp.dot(a, b, preferred_element_type=jnp.float32)

    mask = (adj_ref[...] != 0.0).astype(jnp.float32)
    maskT = (adjT_ref[...] != 0.0).astype(jnp.float32)
    n_edges = jnp.sum(mask)
    fill = jnp.float32(N * N) - n_edges
    ii = jax.lax.broadcasted_iota(jnp.int32, (N, 1), 0)
    jj = jax.lax.broadcasted_iota(jnp.int32, (1, N), 1)
    indeg_col = jnp.sum(maskT, axis=1, keepdims=True)
    indeg_row = jnp.sum(mask, axis=0, keepdims=True)
    deg_col = jnp.maximum(indeg_col + jnp.where(ii == 0, fill, 0.0), 1.0)
    deg_row = jnp.maximum(indeg_row + jnp.where(jj == 0, fill, 0.0), 1.0)
    r_col = jax.lax.rsqrt(deg_col)
    r_row = jax.lax.rsqrt(deg_row)
    deg0 = jnp.sum(jnp.where(ii == 0, deg_col, 0.0))
    corr = fill / deg0
    ahat = maskT * (r_col * r_row)
    ahat = ahat + jnp.where((ii == 0) & (jj == 0), corr, 0.0)

    b1 = b1_ref[...]
    bt = bt_ref[...]
    gamma = gamma_ref[...]
    beta = beta_ref[...]

    for b in range(B):
        feat = dot(xt_ref[b].reshape(T * N, D), w1_ref[...]) + b1
        featbuf[b] = feat.reshape(T, N, D)
        q0 = dot(feat, th0_ref[...])
        q1 = dot(feat, th1_ref[...]).reshape(T, N, F)
        q0 = q0.reshape(T, N, F)
        sp_pad[b, 0] = jnp.zeros((N, F), jnp.float32)
        sp_pad[b, T + 1] = jnp.zeros((N, F), jnp.float32)
        for t in range(T):
            agg = dot(ahat, q1[t])
            sp_pad[b, t + 1] = jnp.maximum(q0[t] + agg, 0.0)

    for b in range(B):
        conv = dot(sp_pad[b, 0:T].reshape(T * N, F), wtT_ref[0])
        conv += dot(sp_pad[b, 1:T + 1].reshape(T * N, F), wtT_ref[1])
        conv += dot(sp_pad[b, 2:T + 2].reshape(T * N, F), wtT_ref[2])
        res = dot(featbuf[b].reshape(T * N, D), wr_ref[...])
        out = jnp.maximum(conv + bt + res, 0.0)
        mu = jnp.mean(out, axis=1, keepdims=True)
        cent = out - mu
        var = jnp.mean(cent * cent, axis=1, keepdims=True)
        o = cent * jax.lax.rsqrt(var + 1e-5) * gamma + beta
        out_ref[b] = o.reshape(T, N, F)


def kernel(x, adj, W1, b1, theta0, theta1, Wt, bt, Wr, ln_gamma, ln_beta):
    B, N, T, D = x.shape
    F = theta0.shape[1]
    xt = jnp.transpose(x, (0, 2, 1, 3))
    adjT = jnp.transpose(adj, (1, 0))
    wtT = jnp.transpose(Wt, (2, 1, 0))
    out = pl.pallas_call(
        _encoder_body,
        out_shape=jax.ShapeDtypeStruct((B, T, N, F), jnp.float32),
        scratch_shapes=[
            pltpu.VMEM((B, T + 2, N, F), jnp.float32),
            pltpu.VMEM((B, T, N, D), jnp.float32),
        ],
    )(xt, adj, adjT, W1, b1.reshape(1, -1), theta0, theta1,
      wtT, bt.reshape(1, -1), Wr, ln_gamma.reshape(1, -1), ln_beta.reshape(1, -1))
    return jnp.transpose(out, (0, 2, 3, 1))
